# baseline (speedup 1.0000x reference)
import functools

import jax
import jax.numpy as jnp
from jax import lax
from jax.experimental import pallas as pl
from jax.experimental.pallas import tpu as pltpu


def _prep_kernel(mask_ref, prob_ref, counts_ref,
                 idx_ref, pe_ref, valid_ref, lstar_ref, ntrue_ref, *, L):
    mf = mask_ref[...].astype(jnp.float32)
    p = prob_ref[...]
    cntf = counts_ref[...].astype(jnp.float32)
    B = mf.shape[0]

    c1 = mf
    s = 1
    while s < L:
        shifted = jnp.concatenate(
            [jnp.zeros((B, s), jnp.float32), c1[:, :L - s]], axis=1)
        c1 = c1 + shifted
        s *= 2

    idx_ref[...] = jnp.maximum(c1 - 1.0, 0.0).astype(jnp.int32)
    decay = jnp.clip(1.0 - p, 0.0, 1.0)
    pe_ref[...] = mf * (1.0 - decay)
    valid_ref[...] = jnp.where((c1 >= 1.0) & (c1 <= cntf), 1.0, 0.0)
    lstar = jnp.sum((c1 < cntf).astype(jnp.float32), axis=1, keepdims=True)
    lstar_ref[...] = jnp.minimum(lstar, jnp.float32(L - 1)).astype(jnp.int32)
    ntrue_ref[...] = c1[:, L - 1:L].astype(jnp.int32)


def _scan_kernel(idx_ref, pe_ref, valid_ref, counts_ref, lstar_ref, ntrue_ref,
                 hs_ref, res_ref, state_ref, out_ref, ns_ref, *, L):
    z0 = state_ref[0]

    def body(l, z):
        i = idx_ref[0, 0, l]
        pe = pe_ref[0, 0, l]
        vf = valid_ref[0, 0, l]
        g = hs_ref[0, pl.ds(i, 1)]
        z1 = (1.0 - pe) * z + pe * g
        out_ref[0, pl.ds(l, 1)] = res_ref[0, pl.ds(l, 1)] + vf * z1
        return z1

    z_fin = lax.fori_loop(0, L, body, z0, unroll=16)

    cnt = counts_ref[0, 0, 0]
    ntrue = ntrue_ref[0, 0, 0]
    ls = lstar_ref[0, 0, 0]
    diff = out_ref[0, pl.ds(ls, 1)] - res_ref[0, pl.ds(ls, 1)]
    ns = jnp.where(cnt > ntrue, z_fin, diff)
    ns_ref[0] = jnp.where(cnt == 0, state_ref[0], ns)


def _detok_impl(hidden_states, residual, token_mask, prob, counts, state):
    B, L, D = residual.shape
    M = hidden_states.shape[1]
    mask_i32 = token_mask.astype(jnp.int32)
    counts_2d = counts.astype(jnp.int32).reshape(B, 1)

    idx, pe, valid, lstar, ntrue = pl.pallas_call(
        functools.partial(_prep_kernel, L=L),
        in_specs=[
            pl.BlockSpec((B, L), lambda: (0, 0)),
            pl.BlockSpec((B, L), lambda: (0, 0)),
            pl.BlockSpec((B, 1), lambda: (0, 0)),
        ],
        out_specs=[
            pl.BlockSpec((B, L), lambda: (0, 0)),
            pl.BlockSpec((B, L), lambda: (0, 0)),
            pl.BlockSpec((B, L), lambda: (0, 0)),
            pl.BlockSpec((B, 1), lambda: (0, 0)),
            pl.BlockSpec((B, 1), lambda: (0, 0)),
        ],
        out_shape=[
            jax.ShapeDtypeStruct((B, L), jnp.int32),
            jax.ShapeDtypeStruct((B, L), jnp.float32),
            jax.ShapeDtypeStruct((B, L), jnp.float32),
            jax.ShapeDtypeStruct((B, 1), jnp.int32),
            jax.ShapeDtypeStruct((B, 1), jnp.int32),
        ],
    )(mask_i32, prob, counts_2d)

    out, new_state = pl.pallas_call(
        functools.partial(_scan_kernel, L=L),
        grid=(B,),
        in_specs=[
            pl.BlockSpec((1, 1, L), lambda b: (b, 0, 0), memory_space=pltpu.SMEM),
            pl.BlockSpec((1, 1, L), lambda b: (b, 0, 0), memory_space=pltpu.SMEM),
            pl.BlockSpec((1, 1, L), lambda b: (b, 0, 0), memory_space=pltpu.SMEM),
            pl.BlockSpec((1, 1, 1), lambda b: (b, 0, 0), memory_space=pltpu.SMEM),
            pl.BlockSpec((1, 1, 1), lambda b: (b, 0, 0), memory_space=pltpu.SMEM),
            pl.BlockSpec((1, 1, 1), lambda b: (b, 0, 0), memory_space=pltpu.SMEM),
            pl.BlockSpec((1, M, D), lambda b: (b, 0, 0)),
            pl.BlockSpec((1, L, D), lambda b: (b, 0, 0)),
            pl.BlockSpec((1, 1, D), lambda b: (b, 0, 0)),
        ],
        out_specs=[
            pl.BlockSpec((1, L, D), lambda b: (b, 0, 0)),
            pl.BlockSpec((1, 1, D), lambda b: (b, 0, 0)),
        ],
        out_shape=[
            jax.ShapeDtypeStruct((B, L, D), jnp.float32),
            jax.ShapeDtypeStruct((B, 1, D), jnp.float32),
        ],
        compiler_params=pltpu.CompilerParams(
            dimension_semantics=("arbitrary",),
            vmem_limit_bytes=64 * 1024 * 1024,
        ),
    )(idx.reshape(B, 1, L), pe.reshape(B, 1, L), valid.reshape(B, 1, L),
      counts_2d.reshape(B, 1, 1), lstar.reshape(B, 1, 1),
      ntrue.reshape(B, 1, 1), hidden_states, residual,
      state.reshape(B, 1, D))
    return (out, new_state.reshape(B, D))


_tpu_devs = [d for d in jax.devices() if d.platform == "tpu"]
_NCORES = 2 if len(_tpu_devs) >= 2 else 1

if _NCORES > 1:
    import numpy as _np
    from jax.sharding import Mesh as _Mesh
    from jax.sharding import NamedSharding as _NamedSharding
    from jax.sharding import PartitionSpec as _P

    _mesh = _Mesh(_np.array(_tpu_devs[:_NCORES]), ("b",))
    _in_specs = (_P("b", None, None), _P("b", None, None), _P("b", None),
                 _P("b", None), _P("b"), _P("b", None))
    _out_specs = (_P("b", None, None), _P("b", None))
    _impl_sharded = jax.shard_map(
        _detok_impl, mesh=_mesh, in_specs=_in_specs, out_specs=_out_specs,
        check_vma=False)

    def kernel(hidden_states, residual, token_mask, prob, counts, state):
        return _impl_sharded(hidden_states, residual, token_mask, prob,
                             counts, state)

    kernel = jax.jit(
        kernel,
        in_shardings=tuple(_NamedSharding(_mesh, s) for s in _in_specs),
        out_shardings=tuple(_NamedSharding(_mesh, s) for s in _out_specs),
    )
else:
    kernel = jax.jit(_detok_impl)


# speedup vs baseline: 5.1115x; 5.1115x over previous
import functools

import jax
import jax.numpy as jnp
from jax import lax
from jax.experimental import pallas as pl
from jax.experimental.pallas import tpu as pltpu


def _prep_kernel(mask_ref, prob_ref, counts_ref,
                 idx_ref, pe_ref, valid_ref, lstar_ref, ntrue_ref, *, L):
    mf = mask_ref[...].astype(jnp.float32)
    p = prob_ref[...]
    cntf = counts_ref[...].astype(jnp.float32)
    B = mf.shape[0]

    c1 = mf
    s = 1
    while s < L:
        shifted = jnp.concatenate(
            [jnp.zeros((B, s), jnp.float32), c1[:, :L - s]], axis=1)
        c1 = c1 + shifted
        s *= 2

    idx_ref[...] = jnp.maximum(c1 - 1.0, 0.0).astype(jnp.int32)
    decay = jnp.clip(1.0 - p, 0.0, 1.0)
    pe_ref[...] = mf * (1.0 - decay)
    valid_ref[...] = jnp.where((c1 >= 1.0) & (c1 <= cntf), 1.0, 0.0)
    lstar = jnp.sum((c1 < cntf).astype(jnp.float32), axis=1, keepdims=True)
    lstar_ref[...] = jnp.minimum(lstar, jnp.float32(L - 1)).astype(jnp.int32)
    ntrue_ref[...] = c1[:, L - 1:L].astype(jnp.int32)


def _scan_kernel(idx_ref, pe_ref, valid_ref, counts_ref, lstar_ref, ntrue_ref,
                 hs_ref, res_ref, state_ref, out_ref, ns_ref, *, L):
    z0 = state_ref[0]

    def body(l, z):
        i = idx_ref[0, 0, l]
        pe = pe_ref[0, 0, l]
        vf = valid_ref[0, 0, l]
        g = hs_ref[0, pl.ds(i, 1)]
        z1 = (1.0 - pe) * z + pe * g
        out_ref[0, pl.ds(l, 1)] = res_ref[0, pl.ds(l, 1)] + vf * z1
        return z1

    z_fin = lax.fori_loop(0, L, body, z0, unroll=16)

    cnt = counts_ref[0, 0, 0]
    ntrue = ntrue_ref[0, 0, 0]
    ls = lstar_ref[0, 0, 0]
    diff = out_ref[0, pl.ds(ls, 1)] - res_ref[0, pl.ds(ls, 1)]
    ns = jnp.where(cnt > ntrue, z_fin, diff)
    ns_ref[0] = jnp.where(cnt == 0, state_ref[0], ns)


def _detok_impl(hidden_states, residual, token_mask, prob, counts, state):
    B, L, D = residual.shape
    M = hidden_states.shape[1]
    mask_i32 = token_mask.astype(jnp.int32)
    counts_2d = counts.astype(jnp.int32).reshape(B, 1)

    idx, pe, valid, lstar, ntrue = pl.pallas_call(
        functools.partial(_prep_kernel, L=L),
        in_specs=[
            pl.BlockSpec((B, L), lambda: (0, 0)),
            pl.BlockSpec((B, L), lambda: (0, 0)),
            pl.BlockSpec((B, 1), lambda: (0, 0)),
        ],
        out_specs=[
            pl.BlockSpec((B, L), lambda: (0, 0)),
            pl.BlockSpec((B, L), lambda: (0, 0)),
            pl.BlockSpec((B, L), lambda: (0, 0)),
            pl.BlockSpec((B, 1), lambda: (0, 0)),
            pl.BlockSpec((B, 1), lambda: (0, 0)),
        ],
        out_shape=[
            jax.ShapeDtypeStruct((B, L), jnp.int32),
            jax.ShapeDtypeStruct((B, L), jnp.float32),
            jax.ShapeDtypeStruct((B, L), jnp.float32),
            jax.ShapeDtypeStruct((B, 1), jnp.int32),
            jax.ShapeDtypeStruct((B, 1), jnp.int32),
        ],
    )(mask_i32, prob, counts_2d)

    out, new_state = pl.pallas_call(
        functools.partial(_scan_kernel, L=L),
        grid=(B,),
        in_specs=[
            pl.BlockSpec((1, 1, L), lambda b: (b, 0, 0), memory_space=pltpu.SMEM),
            pl.BlockSpec((1, 1, L), lambda b: (b, 0, 0), memory_space=pltpu.SMEM),
            pl.BlockSpec((1, 1, L), lambda b: (b, 0, 0), memory_space=pltpu.SMEM),
            pl.BlockSpec((1, 1, 1), lambda b: (b, 0, 0), memory_space=pltpu.SMEM),
            pl.BlockSpec((1, 1, 1), lambda b: (b, 0, 0), memory_space=pltpu.SMEM),
            pl.BlockSpec((1, 1, 1), lambda b: (b, 0, 0), memory_space=pltpu.SMEM),
            pl.BlockSpec((1, M, D), lambda b: (b, 0, 0)),
            pl.BlockSpec((1, L, D), lambda b: (b, 0, 0)),
            pl.BlockSpec((1, 1, D), lambda b: (b, 0, 0)),
        ],
        out_specs=[
            pl.BlockSpec((1, L, D), lambda b: (b, 0, 0)),
            pl.BlockSpec((1, 1, D), lambda b: (b, 0, 0)),
        ],
        out_shape=[
            jax.ShapeDtypeStruct((B, L, D), jnp.float32),
            jax.ShapeDtypeStruct((B, 1, D), jnp.float32),
        ],
        compiler_params=pltpu.CompilerParams(
            dimension_semantics=("arbitrary",),
            vmem_limit_bytes=64 * 1024 * 1024,
        ),
    )(idx.reshape(B, 1, L), pe.reshape(B, 1, L), valid.reshape(B, 1, L),
      counts_2d.reshape(B, 1, 1), lstar.reshape(B, 1, 1),
      ntrue.reshape(B, 1, 1), hidden_states, residual,
      state.reshape(B, 1, D))
    return (out, new_state.reshape(B, D))


@jax.jit
def kernel(hidden_states, residual, token_mask, prob, counts, state):
    return _detok_impl(hidden_states, residual, token_mask, prob, counts,
                       state)


# speedup vs baseline: 5.2082x; 1.0189x over previous
import functools

import jax
import jax.numpy as jnp
from jax import lax
from jax.experimental import pallas as pl
from jax.experimental.pallas import tpu as pltpu


def _prep_kernel(mask_ref, prob_ref, counts_ref,
                 idx_ref, pe_ref, valid_ref, lstar_ref, ntrue_ref, *, L):
    mf = mask_ref[...].astype(jnp.float32)
    p = prob_ref[...]
    cntf = counts_ref[...].astype(jnp.float32)
    B = mf.shape[0]

    c1 = mf
    s = 1
    while s < L:
        shifted = jnp.concatenate(
            [jnp.zeros((B, s), jnp.float32), c1[:, :L - s]], axis=1)
        c1 = c1 + shifted
        s *= 2

    idx_ref[...] = jnp.maximum(c1 - 1.0, 0.0).astype(jnp.int32)
    decay = jnp.clip(1.0 - p, 0.0, 1.0)
    pe_ref[...] = mf * (1.0 - decay)
    valid_ref[...] = jnp.where((c1 >= 1.0) & (c1 <= cntf), 1.0, 0.0)
    lstar = jnp.sum((c1 < cntf).astype(jnp.float32), axis=1, keepdims=True)
    lstar_ref[...] = jnp.minimum(lstar, jnp.float32(L - 1)).astype(jnp.int32)
    ntrue_ref[...] = c1[:, L - 1:L].astype(jnp.int32)


def _scan_kernel(idx_ref, pe_ref, valid_ref, counts_ref, lstar_ref, ntrue_ref,
                 hs_ref, res_ref, state_ref, out_ref, ns_ref, *, L):
    z0 = state_ref[0]

    def body(l, z):
        i = idx_ref[0, 0, l]
        pe = pe_ref[0, 0, l]
        vf = valid_ref[0, 0, l]
        g = hs_ref[0, pl.ds(i, 1)]
        z1 = (1.0 - pe) * z + pe * g
        out_ref[0, pl.ds(l, 1)] = res_ref[0, pl.ds(l, 1)] + vf * z1
        return z1

    z_fin = lax.fori_loop(0, L, body, z0, unroll=4)

    cnt = counts_ref[0, 0, 0]
    ntrue = ntrue_ref[0, 0, 0]
    ls = lstar_ref[0, 0, 0]
    diff = out_ref[0, pl.ds(ls, 1)] - res_ref[0, pl.ds(ls, 1)]
    ns = jnp.where(cnt > ntrue, z_fin, diff)
    ns_ref[0] = jnp.where(cnt == 0, state_ref[0], ns)


def _detok_impl(hidden_states, residual, token_mask, prob, counts, state):
    B, L, D = residual.shape
    M = hidden_states.shape[1]
    mask_i32 = token_mask.astype(jnp.int32)
    counts_2d = counts.astype(jnp.int32).reshape(B, 1)

    idx, pe, valid, lstar, ntrue = pl.pallas_call(
        functools.partial(_prep_kernel, L=L),
        in_specs=[
            pl.BlockSpec((B, L), lambda: (0, 0)),
            pl.BlockSpec((B, L), lambda: (0, 0)),
            pl.BlockSpec((B, 1), lambda: (0, 0)),
        ],
        out_specs=[
            pl.BlockSpec((B, L), lambda: (0, 0)),
            pl.BlockSpec((B, L), lambda: (0, 0)),
            pl.BlockSpec((B, L), lambda: (0, 0)),
            pl.BlockSpec((B, 1), lambda: (0, 0)),
            pl.BlockSpec((B, 1), lambda: (0, 0)),
        ],
        out_shape=[
            jax.ShapeDtypeStruct((B, L), jnp.int32),
            jax.ShapeDtypeStruct((B, L), jnp.float32),
            jax.ShapeDtypeStruct((B, L), jnp.float32),
            jax.ShapeDtypeStruct((B, 1), jnp.int32),
            jax.ShapeDtypeStruct((B, 1), jnp.int32),
        ],
    )(mask_i32, prob, counts_2d)

    out, new_state = pl.pallas_call(
        functools.partial(_scan_kernel, L=L),
        grid=(B,),
        in_specs=[
            pl.BlockSpec((1, 1, L), lambda b: (b, 0, 0), memory_space=pltpu.SMEM),
            pl.BlockSpec((1, 1, L), lambda b: (b, 0, 0), memory_space=pltpu.SMEM),
            pl.BlockSpec((1, 1, L), lambda b: (b, 0, 0), memory_space=pltpu.SMEM),
            pl.BlockSpec((1, 1, 1), lambda b: (b, 0, 0), memory_space=pltpu.SMEM),
            pl.BlockSpec((1, 1, 1), lambda b: (b, 0, 0), memory_space=pltpu.SMEM),
            pl.BlockSpec((1, 1, 1), lambda b: (b, 0, 0), memory_space=pltpu.SMEM),
            pl.BlockSpec((1, M, D), lambda b: (b, 0, 0)),
            pl.BlockSpec((1, L, D), lambda b: (b, 0, 0)),
            pl.BlockSpec((1, 1, D), lambda b: (b, 0, 0)),
        ],
        out_specs=[
            pl.BlockSpec((1, L, D), lambda b: (b, 0, 0)),
            pl.BlockSpec((1, 1, D), lambda b: (b, 0, 0)),
        ],
        out_shape=[
            jax.ShapeDtypeStruct((B, L, D), jnp.float32),
            jax.ShapeDtypeStruct((B, 1, D), jnp.float32),
        ],
        compiler_params=pltpu.CompilerParams(
            dimension_semantics=("arbitrary",),
            vmem_limit_bytes=64 * 1024 * 1024,
        ),
    )(idx.reshape(B, 1, L), pe.reshape(B, 1, L), valid.reshape(B, 1, L),
      counts_2d.reshape(B, 1, 1), lstar.reshape(B, 1, 1),
      ntrue.reshape(B, 1, 1), hidden_states, residual,
      state.reshape(B, 1, D))
    return (out, new_state.reshape(B, D))


@jax.jit
def kernel(hidden_states, residual, token_mask, prob, counts, state):
    return _detok_impl(hidden_states, residual, token_mask, prob, counts,
                       state)
